# baseline (speedup 1.0000x reference)
import functools

import jax
import jax.numpy as jnp
from jax import lax
from jax.experimental import pallas as pl
from jax.experimental.pallas import tpu as pltpu
from jax.experimental.pallas import tpu_sc as plsc

B, C, H, W = 4, 64, 512, 512
N = 512
P = 16
L = 16
WB = W // L
V = B * C * H * W // L
CHW = C * H * W // L
COFF = H * W // L
NPATCH = B * N
RPP = C * P


def _patch_kernel(t0_h, t1_h, t2_h, t3_h, cy_h, cx_h, out_h,
                  cyv, cxv, tmp, bufs, obufs, idxs, sems):
    tables = (t0_h, t1_h, t2_h, t3_h)
    info = plsc.get_sparse_core_info()
    nc = info.num_cores
    nw = nc * info.num_subcores
    ppw = NPATCH // nw

    wid = lax.axis_index("s") * nc + lax.axis_index("c")
    base_p = wid * ppw

    pltpu.sync_copy(cy_h.at[pl.ds(base_p, ppw + 16)], cyv)
    pltpu.sync_copy(cx_h.at[pl.ds(base_p, ppw + 16)], cxv)

    lane = lax.iota(jnp.int32, 16)

    def scalars(k):
        return cyv[pl.ds(k, 16)][0], cxv[pl.ds(k, 16)][0]

    def idx_build(k, par):
        idx = idxs.at[par]
        cy_s, cx_s = scalars(k)
        fx = cx_s - 8
        b0 = fx >> 4
        ry = cy_s - 8 + lane
        ryc = jnp.clip(ry, 0, H - 1)
        v0 = ryc * WB + b0
        tmp[pl.ds(0, 16)] = v0
        w0 = plsc.load_gather(tmp, [lane >> 1]) + (lane & 1)
        w1 = plsc.load_gather(tmp, [(lane >> 1) + 8]) + (lane & 1)

        @plsc.parallel_loop(0, C)
        def cbody(c):
            coff = c * COFF
            idx[c >> 2, pl.ds((c & 3) * 32, 16)] = jnp.clip(
                w0 + coff, 0, CHW - 1)
            idx[c >> 2, pl.ds((c & 3) * 32 + 16, 16)] = jnp.clip(
                w1 + coff, 0, CHW - 1)

    b_img = wid >> 3

    def fire(par):
        buf, idx, sg = bufs.at[par], idxs.at[par], sems.at[par]
        for bb in range(B):

            @pl.when(b_img == bb)
            def _(bb=bb):
                for j in range(16):
                    pltpu.make_async_copy(tables[bb].at[idx.at[j]],
                                          buf.at[pl.ds(j * 128, 128)],
                                          sg).start()

    def body(k, par, prep):
        buf, obuf = bufs.at[par], obufs.at[par]
        idx = idxs.at[par]
        sg, ss = sems.at[par], sems.at[2 + par]
        for j in range(16):
            pltpu.make_async_copy(t0_h.at[idx.at[j]],
                                  buf.at[pl.ds(j * 128, 128)], sg).wait()

        cy_s, cx_s = scalars(k)
        fx = cx_s - 8
        s = fx & 15
        interior = ((cy_s >= 8) & (cy_s <= H - 8)
                    & (cx_s >= 8) & (cx_s <= W - 8))
        civ = s + lane
        rofs = civ >> 4
        cc15 = civ & 15

        @pl.when(interior)
        def _():
            @plsc.parallel_loop(0, C, unroll=2)
            def ibody(c):
                for py in range(16):
                    ridx = rofs + (32 * c + 2 * py)
                    obuf[c, py, :] = plsc.load_gather(buf, [ridx, cc15])

        @pl.when(jnp.logical_not(interior))
        def _():
            colpos = fx + lane
            cmask = (colpos >= 0) & (colpos < W)
            masks = []
            for py in range(16):
                ry_s = cy_s - 8 + py
                rv = (ry_s >= 0) & (ry_s < H)
                masks.append(jnp.logical_and(cmask, rv))

            def ebody(c, _):
                for py in range(16):
                    ridx = rofs + (32 * c + 2 * py)
                    val = plsc.load_gather(buf, [ridx, cc15])
                    obuf[c, py, :] = jnp.where(masks[py], val, 0.0)
                return 0

            lax.fori_loop(0, C, ebody, 0)

        pk = base_p + k
        pltpu.make_async_copy(obuf, out_h.at[pk >> 9, pk & (N - 1)],
                              ss).start()

        if prep:
            idx_build(k + 2, par)
            pltpu.make_async_copy(obuf, out_h.at[pk >> 9, pk & (N - 1)],
                                  ss).wait()
            fire(par)

    idx_build(0, 0)
    fire(0)
    idx_build(1, 1)
    fire(1)

    def pair(k2, carry):
        k = 2 * k2
        body(k, 0, True)
        body(k + 1, 1, True)
        return carry

    lax.fori_loop(0, ppw // 2 - 1, pair, 0)
    body(ppw - 2, 0, False)
    body(ppw - 1, 1, False)
    p2 = base_p + ppw - 2
    p1 = base_p + ppw - 1
    pltpu.make_async_copy(obufs.at[0], out_h.at[p2 >> 9, p2 & (N - 1)],
                          sems.at[2]).wait()
    pltpu.make_async_copy(obufs.at[1], out_h.at[p1 >> 9, p1 & (N - 1)],
                          sems.at[3]).wait()


def _run(tables, cy, cx):
    mesh = plsc.VectorSubcoreMesh(core_axis_name="c", subcore_axis_name="s")
    kfn = functools.partial(
        pl.kernel,
        mesh=mesh,
        out_type=jax.ShapeDtypeStruct((B, N, C, P, P), jnp.float32),
        scratch_types=[
            pltpu.VMEM((NPATCH // 32 + 16,), jnp.int32),
            pltpu.VMEM((NPATCH // 32 + 16,), jnp.int32),
            pltpu.VMEM((128,), jnp.int32),
            pltpu.VMEM((2, 2 * RPP, P), jnp.float32),
            pltpu.VMEM((2, C, P, P), jnp.float32),
            pltpu.VMEM((2, 16, 128), jnp.int32),
            pltpu.SemaphoreType.DMA((4,)),
        ],
        compiler_params=pltpu.CompilerParams(use_tc_tiling_on_sc=False,
                                             needs_layout_passes=False),
    )(_patch_kernel)
    return kfn(*tables, cy, cx)


def kernel(tensor, centers, patch_size):
    del patch_size
    tables = [tensor[b].reshape(CHW, L) for b in range(B)]
    c32 = centers.astype(jnp.int32).reshape(NPATCH, 2)
    pad = jnp.zeros((16,), jnp.int32)
    cy = jnp.concatenate([c32[:, 0], pad])
    cx = jnp.concatenate([c32[:, 1], pad])
    return _run(tables, cy, cx)


# speedup vs baseline: 1.1225x; 1.1225x over previous
import functools

import jax
import jax.numpy as jnp
from jax import lax
from jax.experimental import pallas as pl
from jax.experimental.pallas import tpu as pltpu
from jax.experimental.pallas import tpu_sc as plsc

B, C, H, W = 4, 64, 512, 512
N = 512
P = 16
L = 16
WB = W // L
V = B * C * H * W // L
CHW = C * H * W // L
COFF = H * W // L
NPATCH = B * N
RPP = C * P


def _patch_kernel(table_h, cy_h, cx_h, out_h,
                  cyv, cxv, tmp, bufs, obufs, idxs, sems):
    info = plsc.get_sparse_core_info()
    nc = info.num_cores
    nw = nc * info.num_subcores
    ppw = NPATCH // nw

    wid = lax.axis_index("s") * nc + lax.axis_index("c")
    base_p = wid * ppw

    pltpu.sync_copy(cy_h.at[pl.ds(base_p, ppw + 16)], cyv)
    pltpu.sync_copy(cx_h.at[pl.ds(base_p, ppw + 16)], cxv)

    lane = lax.iota(jnp.int32, 16)

    def scalars(k):
        return cyv[pl.ds(k, 16)][0], cxv[pl.ds(k, 16)][0]

    def idx_build(k, par):
        idx = idxs.at[par]
        cy_s, cx_s = scalars(k)
        fx = cx_s - 8
        b0 = fx >> 4
        b_img = (base_p + k) >> 9
        ry = cy_s - 8 + lane
        ryc = jnp.clip(ry, 0, H - 1)
        v0 = b_img * CHW + ryc * WB + b0
        tmp[pl.ds(0, 16)] = v0
        w0 = plsc.load_gather(tmp, [lane >> 1]) + (lane & 1)
        w1 = plsc.load_gather(tmp, [(lane >> 1) + 8]) + (lane & 1)

        @plsc.parallel_loop(0, C)
        def cbody(c):
            coff = c * COFF
            idx[c >> 2, pl.ds((c & 3) * 32, 16)] = jnp.clip(
                w0 + coff, 0, V - 1)
            idx[c >> 2, pl.ds((c & 3) * 32 + 16, 16)] = jnp.clip(
                w1 + coff, 0, V - 1)

    def fire(par):
        buf, idx, sg = bufs.at[par], idxs.at[par], sems.at[par]
        for j in range(16):
            pltpu.make_async_copy(table_h.at[idx.at[j]],
                                  buf.at[pl.ds(j * 128, 128)], sg).start()

    def body(k, par, prep):
        buf, obuf = bufs.at[par], obufs.at[par]
        idx = idxs.at[par]
        sg, ss = sems.at[par], sems.at[2 + par]
        pltpu.make_async_copy(table_h.at[pl.ds(0, 2 * RPP)], buf, sg).wait()

        cy_s, cx_s = scalars(k)
        fx = cx_s - 8
        s = fx & 15
        interior = ((cy_s >= 8) & (cy_s <= H - 8)
                    & (cx_s >= 8) & (cx_s <= W - 8))
        civ = s + lane
        rofs = civ >> 4
        cc15 = civ & 15

        @pl.when(interior)
        def _():
            @plsc.parallel_loop(0, C, unroll=4)
            def ibody(c):
                for py in range(16):
                    ridx = rofs + (32 * c + 2 * py)
                    obuf[c, py, :] = plsc.load_gather(buf, [ridx, cc15])

        @pl.when(jnp.logical_not(interior))
        def _():
            colpos = fx + lane
            cmask = (colpos >= 0) & (colpos < W)
            masks = []
            for py in range(16):
                ry_s = cy_s - 8 + py
                rv = (ry_s >= 0) & (ry_s < H)
                masks.append(jnp.logical_and(cmask, rv))

            def ebody(c, _):
                for py in range(16):
                    ridx = rofs + (32 * c + 2 * py)
                    val = plsc.load_gather(buf, [ridx, cc15])
                    obuf[c, py, :] = jnp.where(masks[py], val, 0.0)
                return 0

            lax.fori_loop(0, C, ebody, 0)

        pk = base_p + k
        pltpu.make_async_copy(obuf, out_h.at[pk >> 9, pk & (N - 1)],
                              ss).start()

        if prep:
            idx_build(k + 2, par)
            pltpu.make_async_copy(obuf, out_h.at[pk >> 9, pk & (N - 1)],
                                  ss).wait()
            fire(par)

    idx_build(0, 0)
    fire(0)
    idx_build(1, 1)
    fire(1)

    def pair(k2, carry):
        k = 2 * k2
        body(k, 0, True)
        body(k + 1, 1, True)
        return carry

    lax.fori_loop(0, ppw // 2 - 1, pair, 0)
    body(ppw - 2, 0, False)
    body(ppw - 1, 1, False)
    p2 = base_p + ppw - 2
    p1 = base_p + ppw - 1
    pltpu.make_async_copy(obufs.at[0], out_h.at[p2 >> 9, p2 & (N - 1)],
                          sems.at[2]).wait()
    pltpu.make_async_copy(obufs.at[1], out_h.at[p1 >> 9, p1 & (N - 1)],
                          sems.at[3]).wait()


def _run(table, cy, cx):
    mesh = plsc.VectorSubcoreMesh(core_axis_name="c", subcore_axis_name="s")
    kfn = functools.partial(
        pl.kernel,
        mesh=mesh,
        out_type=jax.ShapeDtypeStruct((B, N, C, P, P), jnp.float32),
        scratch_types=[
            pltpu.VMEM((NPATCH // 32 + 16,), jnp.int32),
            pltpu.VMEM((NPATCH // 32 + 16,), jnp.int32),
            pltpu.VMEM((128,), jnp.int32),
            pltpu.VMEM((2, 2 * RPP, P), jnp.float32),
            pltpu.VMEM((2, C, P, P), jnp.float32),
            pltpu.VMEM((2, 16, 128), jnp.int32),
            pltpu.SemaphoreType.DMA((4,)),
        ],
        compiler_params=pltpu.CompilerParams(use_tc_tiling_on_sc=False,
                                             needs_layout_passes=False),
    )(_patch_kernel)
    return kfn(table, cy, cx)


def kernel(tensor, centers, patch_size):
    del patch_size
    table = tensor.reshape(V, L)
    c32 = centers.astype(jnp.int32).reshape(NPATCH, 2)
    pad = jnp.zeros((16,), jnp.int32)
    cy = jnp.concatenate([c32[:, 0], pad])
    cx = jnp.concatenate([c32[:, 1], pad])
    return _run(table, cy, cx)
